# baseline (speedup 1.0000x reference)
import dataclasses

import jax
import jax.numpy as jnp
from jax.experimental import pallas as pl
from jax.experimental.pallas import tpu as pltpu
from jax.experimental.pallas import tpu_sc as plsc

N = 4194304
F = 16
LANES = 16

TC_BLK = 262144
TC_CH = 1024
SC_BLOCK = 8192

N_TC = 9 * TC_BLK
N_SC = N - N_TC

_RECIP_CA = 1.0 / 420.0


def _sc_part(gs0, a1, A, rh, FGs):
    base = N_TC // SC_BLOCK
    mesh = plsc.VectorSubcoreMesh(core_axis_name="core",
                                  subcore_axis_name="subcore")
    cp = pltpu.CompilerParams()
    if "needs_layout_passes" in pltpu.CompilerParams.__dataclass_fields__:
        cp = dataclasses.replace(cp, needs_layout_passes=False)

    @pl.kernel(
        out_type=jax.ShapeDtypeStruct((N_SC,), jnp.float32),
        mesh=mesh,
        compiler_params=cp,
        scratch_types=[pltpu.VMEM((F,), jnp.float32),
                       pltpu.VMEM((F,), jnp.float32)],
    )
    def sc_kernel(gs0_hbm, a1_hbm, a_hbm, rh_hbm, fgs_hbm, out_hbm,
                  gs0_v, a1_v):
        pltpu.sync_copy(gs0_hbm, gs0_v)
        pltpu.sync_copy(a1_hbm, a1_v)
        a1_v[...] = a1_v[...] * _RECIP_CA

        def body(a_vmem, rh_vmem, fgs_vmem, out_vmem):
            @plsc.parallel_loop(0, SC_BLOCK, LANES, unroll=16)
            def _(c):
                sl = pl.ds(c, LANES)
                idx = fgs_vmem[sl]
                g = plsc.load_gather(gs0_v, [idx])
                a = plsc.load_gather(a1_v, [idx])
                out_vmem[sl] = g + a * (a_vmem[sl] * rh_vmem[sl])

        pltpu.emit_pipeline(
            body,
            grid=(N_SC // SC_BLOCK,),
            in_specs=[
                pl.BlockSpec((SC_BLOCK,), lambda i: (base + i,)),
                pl.BlockSpec((SC_BLOCK,), lambda i: (base + i,)),
                pl.BlockSpec((SC_BLOCK,), lambda i: (base + i,)),
            ],
            out_specs=[pl.BlockSpec((SC_BLOCK,), lambda i: (i,))],
            core_axis_name=("core", "subcore"),
            dimension_semantics=(pltpu.PARALLEL,),
        )(a_hbm, rh_hbm, fgs_hbm, out_hbm)

    return sc_kernel(gs0, a1, A, rh, FGs)


def _tc_body(gs0_ref, a1_ref, a_ref, rh_ref, fgs_ref, out_ref):
    tg = gs0_ref[...]
    ta = a1_ref[...]
    tgv = [jnp.full((TC_CH,), tg[f]) for f in range(F)]
    tav = [jnp.full((TC_CH,), ta[f]) for f in range(F)]

    @pl.loop(0, TC_BLK, step=TC_CH, unroll=8)
    def _(c):
        cs = pl.ds(c, TC_CH)
        idx = fgs_ref[cs]
        t = a_ref[cs] * rh_ref[cs] * _RECIP_CA
        b = [(idx & (1 << k)) != 0 for k in range(4)]
        g = [jnp.where(b[0], tgv[2 * i + 1], tgv[2 * i]) for i in range(8)]
        a = [jnp.where(b[0], tav[2 * i + 1], tav[2 * i]) for i in range(8)]
        for k in range(1, 4):
            g = [jnp.where(b[k], g[2 * i + 1], g[2 * i])
                 for i in range(len(g) // 2)]
            a = [jnp.where(b[k], a[2 * i + 1], a[2 * i])
                 for i in range(len(a) // 2)]
        out_ref[cs] = g[0] + a[0] * t


def _tc_part(gs0, a1, A, rh, FGs):
    return pl.pallas_call(
        _tc_body,
        out_shape=jax.ShapeDtypeStruct((N,), jnp.float32),
        grid=(N_TC // TC_BLK,),
        in_specs=[
            pl.BlockSpec((F,), lambda i: (0,)),
            pl.BlockSpec((F,), lambda i: (0,)),
            pl.BlockSpec((TC_BLK,), lambda i: (i,)),
            pl.BlockSpec((TC_BLK,), lambda i: (i,)),
            pl.BlockSpec((TC_BLK,), lambda i: (i,)),
        ],
        out_specs=pl.BlockSpec((TC_BLK,), lambda i: (i,)),
        compiler_params=pltpu.CompilerParams(
            dimension_semantics=("parallel",),
        ),
    )(gs0, a1, A, rh, FGs)


def kernel(gs0, a1, A, rh, FGs):
    sc_out = _sc_part(gs0, a1, A, rh, FGs)
    tc_out = _tc_part(gs0, a1, A, rh, FGs)
    return jax.lax.dynamic_update_slice(tc_out, sc_out, (N_TC,))
